# baseline (speedup 1.0000x reference)
import functools

import jax
import jax.numpy as jnp
from jax.experimental import pallas as pl
from jax.experimental.pallas import tpu as pltpu


def _emb_body(trees_ref, idx_ref, valid_ref, W_ref, gate_ref, rsel_ref,
              bmask_ref, comb_ref, emb_buf, *, BP, F, N):
    f32 = jnp.float32
    bf16 = jnp.bfloat16
    i = pl.program_id(0)
    ib = jax.lax.rem(i, 2)

    emb_prev = emb_buf[1 - ib]
    scores = jnp.dot(gate_ref[...], emb_prev,
                     preferred_element_type=f32)
    score_mat = jnp.concatenate(
        [scores[:, p * N:(p + 1) * N] for p in range(BP)], axis=0)
    v = valid_ref[...]
    score_mat = jnp.where(v > 0.5, score_mat, -1e30)
    m = jnp.max(score_mat, axis=1, keepdims=True)
    e = jnp.exp(score_mat - m) * v
    denom = jnp.sum(e, axis=1, keepdims=True)
    w_bf = (e / denom).astype(bf16)

    w_tiled = jnp.concatenate([w_bf] * BP, axis=1)
    w_sel = w_tiled * bmask_ref[...]
    SW = jnp.concatenate([rsel_ref[...], w_sel], axis=0)
    R = jax.lax.dot_general(SW, emb_prev, (((1,), (1,)), ((), ())),
                            preferred_element_type=f32)
    comb_ref[...] = jnp.concatenate([R[0:BP], R[BP:2 * BP]], axis=1)

    W = W_ref[...]
    for p in range(BP):
        T = trees_ref[p]
        idxb = jnp.broadcast_to(idx_ref[p:p + 1, :], (F, N))
        G = jnp.take_along_axis(T, idxb, axis=1, mode="promise_in_bounds")
        T2 = (T + G).astype(bf16)
        emb_buf[ib, :, p * N:(p + 1) * N] = jnp.maximum(
            jnp.dot(W, T2, preferred_element_type=f32).astype(bf16), 0)


def _mlp_body(comb_ref, W1_ref, b1_ref, W2_ref, b2_ref, out_ref):
    f32 = jnp.float32
    bf16 = jnp.bfloat16
    h = jnp.maximum(
        jnp.dot(comb_ref[...].astype(bf16), W1_ref[...],
                preferred_element_type=f32) + b1_ref[...], 0.0)
    out_ref[...] = jnp.dot(h.astype(bf16), W2_ref[...],
                           preferred_element_type=f32) + b2_ref[...]


def kernel(trees, indexes, mask_padding, W_emb, gate, W1, b1, W2, b2):
    P, F, N = trees.shape
    H = W1.shape[1]
    BP = 128
    NS = P // BP
    f32 = jnp.float32
    bf16 = jnp.bfloat16

    valid = 1.0 - mask_padding.astype(f32)
    W_bf = W_emb.astype(bf16)
    gate2 = gate.reshape(1, F).astype(bf16)

    lane = jnp.arange(BP * N, dtype=jnp.int32)[None, :]
    sub = jnp.arange(BP, dtype=jnp.int32)[:, None]
    rsel = (lane == sub * N + 1).astype(bf16)
    bmask = (lane // N == sub).astype(bf16)

    comb = pl.pallas_call(
        functools.partial(_emb_body, BP=BP, F=F, N=N),
        grid=(NS + 1,),
        in_specs=[
            pl.BlockSpec((BP, F, N),
                         lambda i: (jnp.minimum(i, NS - 1), 0, 0)),
            pl.BlockSpec((BP, N),
                         lambda i: (jnp.minimum(i, NS - 1), 0)),
            pl.BlockSpec((BP, N),
                         lambda i: (jnp.maximum(i - 1, 0), 0)),
            pl.BlockSpec((F, F), lambda i: (0, 0)),
            pl.BlockSpec((1, F), lambda i: (0, 0)),
            pl.BlockSpec((BP, BP * N), lambda i: (0, 0)),
            pl.BlockSpec((BP, BP * N), lambda i: (0, 0)),
        ],
        out_specs=pl.BlockSpec((BP, 2 * F), lambda i: (i, 0)),
        out_shape=jax.ShapeDtypeStruct((P + BP, 2 * F), f32),
        scratch_shapes=[pltpu.VMEM((2, F, BP * N), bf16)],
    )(trees, indexes, valid, W_bf, gate2, rsel, bmask)
    comb = comb[BP:]

    RB = 512
    out = pl.pallas_call(
        _mlp_body,
        grid=(P // RB,),
        in_specs=[
            pl.BlockSpec((RB, 2 * F), lambda i: (i, 0)),
            pl.BlockSpec((2 * F, H), lambda i: (0, 0)),
            pl.BlockSpec((1, H), lambda i: (0, 0)),
            pl.BlockSpec((H, 1), lambda i: (0, 0)),
            pl.BlockSpec((1, 1), lambda i: (0, 0)),
        ],
        out_specs=pl.BlockSpec((RB, 1), lambda i: (i, 0)),
        out_shape=jax.ShapeDtypeStruct((P, 1), f32),
    )(comb, W1.astype(bf16), b1.reshape(1, H), W2.astype(bf16),
      b2.reshape(1, 1))
    return (out, comb)


# speedup vs baseline: 1.1076x; 1.1076x over previous
import functools

import jax
import jax.numpy as jnp
from jax.experimental import pallas as pl
from jax.experimental.pallas import tpu as pltpu


def _emb_body(trees_ref, idx_ref, valid_ref, W_ref, gate_ref, rsel_ref,
              bmask_ref, comb_ref, emb_buf, *, BP, F, N):
    f32 = jnp.float32
    bf16 = jnp.bfloat16
    i = pl.program_id(0)
    ib = jax.lax.rem(i, 2)

    emb_prev = emb_buf[1 - ib]
    scores = jnp.dot(gate_ref[...], emb_prev,
                     preferred_element_type=f32)
    score_mat = jnp.concatenate(
        [scores[:, p * N:(p + 1) * N] for p in range(BP)], axis=0)
    v = valid_ref[...]
    score_mat = jnp.where(v > 0.5, score_mat, -1e30)
    m = jnp.max(score_mat, axis=1, keepdims=True)
    e = jnp.exp(score_mat - m) * v
    denom = jnp.sum(e, axis=1, keepdims=True)
    w_bf = (e / denom).astype(bf16)

    w_tiled = jnp.concatenate([w_bf] * BP, axis=1)
    w_sel = w_tiled * bmask_ref[...]
    SW = jnp.concatenate([rsel_ref[...], w_sel], axis=0)
    R = jax.lax.dot_general(SW, emb_prev, (((1,), (1,)), ((), ())),
                            preferred_element_type=f32)
    comb_ref[...] = jnp.concatenate([R[0:BP], R[BP:2 * BP]], axis=1)

    W = W_ref[...]
    for p in range(BP):
        T = trees_ref[p]
        idxb = jnp.broadcast_to(idx_ref[p:p + 1, :], (F, N))
        G = jnp.take_along_axis(T, idxb, axis=1, mode="promise_in_bounds")
        T2 = (T + G).astype(bf16)
        emb_buf[ib, :, p * N:(p + 1) * N] = jnp.maximum(
            jnp.dot(W, T2, preferred_element_type=f32).astype(bf16), 0)


def _mlp_body(comb_ref, W1_ref, b1_ref, W2_ref, b2_ref, out_ref):
    f32 = jnp.float32
    bf16 = jnp.bfloat16
    h = jnp.maximum(
        jnp.dot(comb_ref[...].astype(bf16), W1_ref[...],
                preferred_element_type=f32) + b1_ref[...], 0.0)
    out_ref[...] = jnp.dot(h.astype(bf16), W2_ref[...],
                           preferred_element_type=f32) + b2_ref[...]


def kernel(trees, indexes, mask_padding, W_emb, gate, W1, b1, W2, b2):
    P, F, N = trees.shape
    H = W1.shape[1]
    BP = 64
    NS = P // BP
    f32 = jnp.float32
    bf16 = jnp.bfloat16

    valid = 1.0 - mask_padding.astype(f32)
    W_bf = W_emb.astype(bf16)
    gate2 = gate.reshape(1, F).astype(bf16)

    lane = jnp.arange(BP * N, dtype=jnp.int32)[None, :]
    sub = jnp.arange(BP, dtype=jnp.int32)[:, None]
    rsel = (lane == sub * N + 1).astype(bf16)
    bmask = (lane // N == sub).astype(bf16)

    comb = pl.pallas_call(
        functools.partial(_emb_body, BP=BP, F=F, N=N),
        grid=(NS + 1,),
        in_specs=[
            pl.BlockSpec((BP, F, N),
                         lambda i: (jnp.minimum(i, NS - 1), 0, 0)),
            pl.BlockSpec((BP, N),
                         lambda i: (jnp.minimum(i, NS - 1), 0)),
            pl.BlockSpec((BP, N),
                         lambda i: (jnp.maximum(i - 1, 0), 0)),
            pl.BlockSpec((F, F), lambda i: (0, 0)),
            pl.BlockSpec((1, F), lambda i: (0, 0)),
            pl.BlockSpec((BP, BP * N), lambda i: (0, 0)),
            pl.BlockSpec((BP, BP * N), lambda i: (0, 0)),
        ],
        out_specs=pl.BlockSpec((BP, 2 * F), lambda i: (i, 0)),
        out_shape=jax.ShapeDtypeStruct((P + BP, 2 * F), f32),
        scratch_shapes=[pltpu.VMEM((2, F, BP * N), bf16)],
    )(trees, indexes, valid, W_bf, gate2, rsel, bmask)
    comb = comb[BP:]

    RB = 512
    out = pl.pallas_call(
        _mlp_body,
        grid=(P // RB,),
        in_specs=[
            pl.BlockSpec((RB, 2 * F), lambda i: (i, 0)),
            pl.BlockSpec((2 * F, H), lambda i: (0, 0)),
            pl.BlockSpec((1, H), lambda i: (0, 0)),
            pl.BlockSpec((H, 1), lambda i: (0, 0)),
            pl.BlockSpec((1, 1), lambda i: (0, 0)),
        ],
        out_specs=pl.BlockSpec((RB, 1), lambda i: (i, 0)),
        out_shape=jax.ShapeDtypeStruct((P, 1), f32),
    )(comb, W1.astype(bf16), b1.reshape(1, H), W2.astype(bf16),
      b2.reshape(1, 1))
    return (out, comb)
